# baseline (speedup 1.0000x reference)
import dataclasses

import jax
import jax.numpy as jnp
from jax.experimental import pallas as pl
from jax.experimental.pallas import tpu as pltpu
from jax.experimental.pallas import tpu_sc as plsc

N = 10000
E = 320000
D = 128
NUM_CORES = 2
NUM_SUBCORES = 16
EDGES_PER_CORE = E // NUM_CORES
EDGES_PER_SUB = EDGES_PER_CORE // NUM_SUBCORES
CHUNK = 80
NUM_CHUNKS = EDGES_PER_SUB // CHUNK
NBUF = 2
N_PAD = 10240
ROWS_PER_SUB = N_PAD // NUM_SUBCORES
ZCHUNK = 32

_VEC = 16


def _sc_agg_body(x_hbm, src_hbm, dst_hbm, agg_hbm, cnt_hbm, table, cnt_slots,
                 src_v, dst_v, rows_v, zero_v, cnt_local, acc_v, tmp_v, sems):
    cid = jax.lax.axis_index("core")
    sid = jax.lax.axis_index("subcore")
    row0 = sid * ROWS_PER_SUB

    @pl.loop(0, ZCHUNK)
    def _(r):
        @pl.loop(0, D, step=_VEC)
        def _(c):
            zero_v[r, pl.ds(c, _VEC)] = jnp.zeros((_VEC,), jnp.float32)

    @pl.loop(0, ROWS_PER_SUB // ZCHUNK)
    def _(k):
        pltpu.sync_copy(zero_v, table.at[pl.ds(row0 + k * ZCHUNK, ZCHUNK)])

    @pl.loop(0, N_PAD, step=_VEC)
    def _(i):
        cnt_local[pl.ds(i, _VEC)] = jnp.zeros((_VEC,), jnp.float32)
    plsc.subcore_barrier()

    base = cid * EDGES_PER_CORE + sid * EDGES_PER_SUB
    ones16 = jnp.ones((_VEC,), jnp.float32)

    def load_and_fire(b, c):
        off = base + c * CHUNK
        pltpu.sync_copy(src_hbm.at[pl.ds(off, CHUNK)], src_v[b])
        pltpu.sync_copy(dst_hbm.at[pl.ds(off, CHUNK)], dst_v[b])
        pltpu.async_copy(x_hbm.at[src_v[b]], rows_v[b], sems[b])

    for b in range(NBUF):
        load_and_fire(b, b)

    @pl.loop(0, (NUM_CHUNKS + NBUF - 1) // NBUF)
    def _(p):
        for b in range(NBUF):
            c = p * NBUF + b

            @pl.when(c < NUM_CHUNKS)
            def _():
                @pl.loop(0, CHUNK, step=_VEC)
                def _(k):
                    plsc.addupdate_scatter(cnt_local, [dst_v[b][pl.ds(k, _VEC)]],
                                           ones16)
                pltpu.make_async_copy(x_hbm.at[src_v[b]], rows_v[b],
                                      sems[b]).wait()
                pltpu.sync_copy(rows_v[b], table.at[dst_v[b]], add=True)

                @pl.when(c + NBUF < NUM_CHUNKS)
                def _():
                    load_and_fire(b, c + NBUF)

    pltpu.sync_copy(cnt_local, cnt_slots.at[sid])
    plsc.subcore_barrier()

    @pl.loop(0, ROWS_PER_SUB, step=_VEC)
    def _(i):
        acc_v[pl.ds(i, _VEC)] = jnp.zeros((_VEC,), jnp.float32)

    @pl.loop(0, NUM_SUBCORES)
    def _(j):
        pltpu.sync_copy(cnt_slots.at[j].at[pl.ds(row0, ROWS_PER_SUB)], tmp_v)

        @pl.loop(0, ROWS_PER_SUB, step=_VEC)
        def _(i):
            acc_v[pl.ds(i, _VEC)] = acc_v[pl.ds(i, _VEC)] + tmp_v[pl.ds(i, _VEC)]

    pltpu.sync_copy(acc_v,
                    cnt_hbm.at[pl.ds(cid * N_PAD + row0, ROWS_PER_SUB)])
    pltpu.sync_copy(table.at[pl.ds(row0, ROWS_PER_SUB)],
                    agg_hbm.at[cid].at[pl.ds(row0, ROWS_PER_SUB)])


_sc_compiler_params = pltpu.CompilerParams()
if "needs_layout_passes" in pltpu.CompilerParams.__dataclass_fields__:
    _sc_compiler_params = dataclasses.replace(_sc_compiler_params,
                                              needs_layout_passes=False)

_sc_agg = pl.kernel(
    _sc_agg_body,
    compiler_params=_sc_compiler_params,
    out_type=(jax.ShapeDtypeStruct((NUM_CORES, N_PAD, D), jnp.float32),
              jax.ShapeDtypeStruct((NUM_CORES * N_PAD,), jnp.float32)),
    mesh=plsc.VectorSubcoreMesh(core_axis_name="core",
                                subcore_axis_name="subcore"),
    scratch_types=(
        pltpu.VMEM_SHARED((N_PAD, D), jnp.float32),
        pltpu.VMEM_SHARED((NUM_SUBCORES, N_PAD), jnp.float32),
        [pltpu.VMEM((CHUNK,), jnp.int32)] * NBUF,
        [pltpu.VMEM((CHUNK,), jnp.int32)] * NBUF,
        [pltpu.VMEM((CHUNK, D), jnp.float32)] * NBUF,
        pltpu.VMEM((ZCHUNK, D), jnp.float32),
        pltpu.VMEM((N_PAD,), jnp.float32),
        pltpu.VMEM((ROWS_PER_SUB,), jnp.float32),
        pltpu.VMEM((ROWS_PER_SUB,), jnp.float32),
        [pltpu.SemaphoreType.DMA] * NBUF,
    ),
)


def _tc_layer(has_resid):
    R = 1000

    def body(agg_ref, cnt_ref, x_ref, wl_ref, wr_ref, b_ref, g_ref, be_ref,
             o_ref):
        recip = 1.0 / jnp.maximum(cnt_ref[...], 1.0)
        agg = (agg_ref[0] + agg_ref[1]) * recip
        xv = x_ref[...]
        h = (jnp.dot(agg, wl_ref[...], preferred_element_type=jnp.float32)
             + jnp.dot(xv, wr_ref[...], preferred_element_type=jnp.float32)
             + b_ref[...])
        mu = jnp.mean(h, axis=-1, keepdims=True)
        d = h - mu
        var = jnp.mean(d * d, axis=-1, keepdims=True)
        h = d * jax.lax.rsqrt(var + 1e-5) * g_ref[...] + be_ref[...]
        h = jnp.maximum(h, 0.0)
        if has_resid:
            h = h + xv
        o_ref[...] = h

    return pl.pallas_call(
        body,
        grid=(N // R,),
        in_specs=[
            pl.BlockSpec((NUM_CORES, R, D), lambda i: (0, i, 0)),
            pl.BlockSpec((R, 1), lambda i: (i, 0)),
            pl.BlockSpec((R, D), lambda i: (i, 0)),
            pl.BlockSpec((D, D), lambda i: (0, 0)),
            pl.BlockSpec((D, D), lambda i: (0, 0)),
            pl.BlockSpec((1, D), lambda i: (0, 0)),
            pl.BlockSpec((1, D), lambda i: (0, 0)),
            pl.BlockSpec((1, D), lambda i: (0, 0)),
        ],
        out_specs=pl.BlockSpec((R, D), lambda i: (i, 0)),
        out_shape=jax.ShapeDtypeStruct((N, D), jnp.float32),
    )


_tc_layer0 = _tc_layer(False)
_tc_layer_res = _tc_layer(True)


def kernel(x, edge_index, W_l0, b_l0, W_r0, gamma0, beta0,
           W_l1, b_l1, W_r1, gamma1, beta1,
           W_l2, b_l2, W_r2, gamma2, beta2):
    src = edge_index[0]
    dst = edge_index[1]

    params = [
        (W_l0, b_l0, W_r0, gamma0, beta0),
        (W_l1, b_l1, W_r1, gamma1, beta1),
        (W_l2, b_l2, W_r2, gamma2, beta2),
    ]

    h = x
    cnt_col = None
    for i, (wl, b, wr, g, be) in enumerate(params):
        agg, cnt_flat = _sc_agg(h, src, dst)
        if i == 0:
            cnt2 = cnt_flat.reshape(NUM_CORES, N_PAD)
            cnt_col = (cnt2[0] + cnt2[1]).reshape(N_PAD, 1)
        tc = _tc_layer0 if i == 0 else _tc_layer_res
        h = tc(agg, cnt_col, h, wl, wr,
               b.reshape(1, D), g.reshape(1, D), be.reshape(1, D))
    return h


# speedup vs baseline: 1.2297x; 1.2297x over previous
import dataclasses

import jax
import jax.numpy as jnp
from jax.experimental import pallas as pl
from jax.experimental.pallas import tpu as pltpu
from jax.experimental.pallas import tpu_sc as plsc

N = 10000
E = 320000
D = 128
NUM_CORES = 2
NUM_SUBCORES = 16
EDGES_PER_CORE = E // NUM_CORES
EDGES_PER_SUB = EDGES_PER_CORE // NUM_SUBCORES
CHUNK = 80
NUM_CHUNKS = EDGES_PER_SUB // CHUNK
NBUF = 3
N_PAD = 10240
ROWS_PER_SUB = N_PAD // NUM_SUBCORES
ZCHUNK = 32

_VEC = 16


def _sc_agg_body(x_hbm, src_hbm, dst_hbm, agg_hbm, cnt_hbm, parts_hbm, table,
                 src_v, dst_v, rows_v, zero_v, cnt_local, acc_v, tmp_v,
                 gsems, ssems):
    cid = jax.lax.axis_index("core")
    sid = jax.lax.axis_index("subcore")
    row0 = sid * ROWS_PER_SUB

    @pl.loop(0, ZCHUNK)
    def _(r):
        @pl.loop(0, D, step=_VEC)
        def _(c):
            zero_v[r, pl.ds(c, _VEC)] = jnp.zeros((_VEC,), jnp.float32)

    @pl.loop(0, ROWS_PER_SUB // ZCHUNK)
    def _(k):
        pltpu.sync_copy(zero_v, table.at[pl.ds(row0 + k * ZCHUNK, ZCHUNK)])

    @pl.loop(0, N_PAD, step=_VEC)
    def _(i):
        cnt_local[pl.ds(i, _VEC)] = jnp.zeros((_VEC,), jnp.float32)
    plsc.subcore_barrier()

    base = cid * EDGES_PER_CORE + sid * EDGES_PER_SUB
    ones16 = jnp.ones((_VEC,), jnp.float32)

    def load_and_fire(b, c):
        off = base + c * CHUNK
        pltpu.sync_copy(src_hbm.at[pl.ds(off, CHUNK)], src_v[b])
        pltpu.sync_copy(dst_hbm.at[pl.ds(off, CHUNK)], dst_v[b])
        pltpu.async_copy(x_hbm.at[src_v[b]], rows_v[b], gsems[b])

    for b in range(NBUF):
        load_and_fire(b, b)

    @pl.loop(0, (NUM_CHUNKS + NBUF - 1) // NBUF)
    def _(p):
        for b in range(NBUF):
            c = p * NBUF + b

            @pl.when(c < NUM_CHUNKS)
            def _():
                @pl.loop(0, CHUNK, step=_VEC)
                def _(k):
                    plsc.addupdate_scatter(cnt_local, [dst_v[b][pl.ds(k, _VEC)]],
                                           ones16)
                pltpu.make_async_copy(x_hbm.at[src_v[b]], rows_v[b],
                                      gsems[b]).wait()
                pltpu.async_copy(rows_v[b], table.at[dst_v[b]], ssems[b],
                                 add=True)

                @pl.when(c + NBUF < NUM_CHUNKS)
                def _():
                    pltpu.make_async_copy(rows_v[b], table.at[dst_v[b]],
                                          ssems[b]).wait()
                    load_and_fire(b, c + NBUF)

    wid = cid * NUM_SUBCORES + sid
    pltpu.sync_copy(cnt_local, parts_hbm.at[pl.ds(wid * N_PAD, N_PAD)])
    plsc.subcore_barrier()

    @pl.loop(0, ROWS_PER_SUB, step=_VEC)
    def _(i):
        acc_v[pl.ds(i, _VEC)] = jnp.zeros((_VEC,), jnp.float32)

    @pl.loop(0, NUM_SUBCORES)
    def _(j):
        pltpu.sync_copy(
            parts_hbm.at[pl.ds((cid * NUM_SUBCORES + j) * N_PAD + row0,
                               ROWS_PER_SUB)], tmp_v)

        @pl.loop(0, ROWS_PER_SUB, step=_VEC)
        def _(i):
            acc_v[pl.ds(i, _VEC)] = acc_v[pl.ds(i, _VEC)] + tmp_v[pl.ds(i, _VEC)]

    pltpu.sync_copy(acc_v,
                    cnt_hbm.at[pl.ds(cid * N_PAD + row0, ROWS_PER_SUB)])
    pltpu.sync_copy(table.at[pl.ds(row0, ROWS_PER_SUB)],
                    agg_hbm.at[cid].at[pl.ds(row0, ROWS_PER_SUB)])


_sc_compiler_params = pltpu.CompilerParams()
if "needs_layout_passes" in pltpu.CompilerParams.__dataclass_fields__:
    _sc_compiler_params = dataclasses.replace(_sc_compiler_params,
                                              needs_layout_passes=False)

_sc_agg = pl.kernel(
    _sc_agg_body,
    compiler_params=_sc_compiler_params,
    out_type=(jax.ShapeDtypeStruct((NUM_CORES, N_PAD, D), jnp.float32),
              jax.ShapeDtypeStruct((NUM_CORES * N_PAD,), jnp.float32),
              jax.ShapeDtypeStruct((NUM_CORES * NUM_SUBCORES * N_PAD,),
                                   jnp.float32)),
    mesh=plsc.VectorSubcoreMesh(core_axis_name="core",
                                subcore_axis_name="subcore"),
    scratch_types=(
        pltpu.VMEM_SHARED((N_PAD, D), jnp.float32),
        [pltpu.VMEM((CHUNK,), jnp.int32)] * NBUF,
        [pltpu.VMEM((CHUNK,), jnp.int32)] * NBUF,
        [pltpu.VMEM((CHUNK, D), jnp.float32)] * NBUF,
        pltpu.VMEM((ZCHUNK, D), jnp.float32),
        pltpu.VMEM((N_PAD,), jnp.float32),
        pltpu.VMEM((ROWS_PER_SUB,), jnp.float32),
        pltpu.VMEM((ROWS_PER_SUB,), jnp.float32),
        [pltpu.SemaphoreType.DMA] * NBUF,
        [pltpu.SemaphoreType.DMA] * NBUF,
    ),
)


def _tc_layer(has_resid):
    R = 1000

    def body(agg_ref, cnt_ref, x_ref, wl_ref, wr_ref, b_ref, g_ref, be_ref,
             o_ref):
        recip = 1.0 / jnp.maximum(cnt_ref[...], 1.0)
        agg = (agg_ref[0] + agg_ref[1]) * recip
        xv = x_ref[...]
        h = (jnp.dot(agg, wl_ref[...], preferred_element_type=jnp.float32)
             + jnp.dot(xv, wr_ref[...], preferred_element_type=jnp.float32)
             + b_ref[...])
        mu = jnp.mean(h, axis=-1, keepdims=True)
        d = h - mu
        var = jnp.mean(d * d, axis=-1, keepdims=True)
        h = d * jax.lax.rsqrt(var + 1e-5) * g_ref[...] + be_ref[...]
        h = jnp.maximum(h, 0.0)
        if has_resid:
            h = h + xv
        o_ref[...] = h

    return pl.pallas_call(
        body,
        grid=(N // R,),
        in_specs=[
            pl.BlockSpec((NUM_CORES, R, D), lambda i: (0, i, 0)),
            pl.BlockSpec((R, 1), lambda i: (i, 0)),
            pl.BlockSpec((R, D), lambda i: (i, 0)),
            pl.BlockSpec((D, D), lambda i: (0, 0)),
            pl.BlockSpec((D, D), lambda i: (0, 0)),
            pl.BlockSpec((1, D), lambda i: (0, 0)),
            pl.BlockSpec((1, D), lambda i: (0, 0)),
            pl.BlockSpec((1, D), lambda i: (0, 0)),
        ],
        out_specs=pl.BlockSpec((R, D), lambda i: (i, 0)),
        out_shape=jax.ShapeDtypeStruct((N, D), jnp.float32),
    )


_tc_layer0 = _tc_layer(False)
_tc_layer_res = _tc_layer(True)


def kernel(x, edge_index, W_l0, b_l0, W_r0, gamma0, beta0,
           W_l1, b_l1, W_r1, gamma1, beta1,
           W_l2, b_l2, W_r2, gamma2, beta2):
    src = edge_index[0]
    dst = edge_index[1]

    params = [
        (W_l0, b_l0, W_r0, gamma0, beta0),
        (W_l1, b_l1, W_r1, gamma1, beta1),
        (W_l2, b_l2, W_r2, gamma2, beta2),
    ]

    h = x
    cnt_col = None
    for i, (wl, b, wr, g, be) in enumerate(params):
        agg, cnt_flat, _ = _sc_agg(h, src, dst)
        if i == 0:
            cnt2 = cnt_flat.reshape(NUM_CORES, N_PAD)
            cnt_col = (cnt2[0] + cnt2[1]).reshape(N_PAD, 1)
        tc = _tc_layer0 if i == 0 else _tc_layer_res
        h = tc(agg, cnt_col, h, wl, wr,
               b.reshape(1, D), g.reshape(1, D), be.reshape(1, D))
    return h
